# Initial kernel scaffold; baseline (speedup 1.0000x reference)
import jax
import jax.numpy as jnp
from jax.experimental import pallas as pl


def kernel(inputs, embedding_matrix):
    raise NotImplementedError("write your pallas kernel here")


# speedup vs baseline: 1.4768x; 1.4768x over previous
import functools

import jax
import jax.numpy as jnp
from jax import lax
from jax.experimental import pallas as pl
from jax.experimental.pallas import tpu as pltpu
from jax.experimental.pallas import tpu_sc as plsc

_EMBED_DIM = 32
_NUM_CORES = 2
_NUM_SUBCORES = 16
_NUM_WORKERS = _NUM_CORES * _NUM_SUBCORES


@functools.partial(jax.jit, static_argnames=("chunk", "n_chunks"))
def _sc_gather(idx, table, *, chunk, n_chunks):
    b_total = idx.shape[0]
    b_per_w = b_total // _NUM_WORKERS
    mesh = plsc.VectorSubcoreMesh(core_axis_name="c", subcore_axis_name="s")

    @functools.partial(
        pl.kernel,
        mesh=mesh,
        out_type=jax.ShapeDtypeStruct((b_total, _EMBED_DIM), jnp.float32),
        scratch_types=[
            pltpu.VMEM((chunk,), jnp.int32),
            pltpu.VMEM((chunk, _EMBED_DIM), jnp.float32),
            pltpu.SemaphoreType.DMA,
        ],
        compiler_params=pltpu.CompilerParams(use_tc_tiling_on_sc=False),
    )
    def k(idx_hbm, table_hbm, out_hbm, idx_v, rows_v, sem):
        wid = lax.axis_index("s") * _NUM_CORES + lax.axis_index("c")
        base = wid * b_per_w

        def body(j, carry):
            off = base + j * chunk
            pltpu.sync_copy(idx_hbm.at[pl.ds(off, chunk)], idx_v)
            pltpu.async_copy(table_hbm.at[idx_v], rows_v, sem).wait()
            pltpu.sync_copy(rows_v, out_hbm.at[pl.ds(off, chunk)])
            return carry

        lax.fori_loop(0, n_chunks, body, 0)

    return k(idx, table)


def kernel(inputs, embedding_matrix):
    batch, hist = inputs.shape
    idx = inputs.reshape(-1).astype(jnp.int32)
    b_per_w = idx.shape[0] // _NUM_WORKERS
    chunk = 1600
    out = _sc_gather(idx, embedding_matrix, chunk=chunk,
                     n_chunks=b_per_w // chunk)
    return out.reshape(batch, hist, _EMBED_DIM)


# speedup vs baseline: 1.4922x; 1.0105x over previous
import functools

import jax
import jax.numpy as jnp
from jax import lax
from jax.experimental import pallas as pl
from jax.experimental.pallas import tpu as pltpu
from jax.experimental.pallas import tpu_sc as plsc

_EMBED_DIM = 32
_NUM_CORES = 2
_NUM_SUBCORES = 16
_NUM_WORKERS = _NUM_CORES * _NUM_SUBCORES


@functools.partial(jax.jit, static_argnames=("chunk", "n_chunks"))
def _sc_gather(idx, table, *, chunk, n_chunks):
    b_total = idx.shape[0]
    b_per_w = b_total // _NUM_WORKERS
    mesh = plsc.VectorSubcoreMesh(core_axis_name="c", subcore_axis_name="s")

    @functools.partial(
        pl.kernel,
        mesh=mesh,
        out_type=jax.ShapeDtypeStruct((b_total, _EMBED_DIM), jnp.float32),
        scratch_types=[
            pltpu.VMEM((chunk,), jnp.int32),
            pltpu.VMEM((chunk,), jnp.int32),
            pltpu.VMEM((chunk, _EMBED_DIM), jnp.float32),
            pltpu.VMEM((chunk, _EMBED_DIM), jnp.float32),
            pltpu.SemaphoreType.DMA,
            pltpu.SemaphoreType.DMA,
            pltpu.SemaphoreType.DMA,
            pltpu.SemaphoreType.DMA,
            pltpu.SemaphoreType.DMA,
            pltpu.SemaphoreType.DMA,
        ],
        compiler_params=pltpu.CompilerParams(use_tc_tiling_on_sc=False),
    )
    def k(idx_hbm, table_hbm, out_hbm,
          idx_v0, idx_v1, rows_v0, rows_v1,
          isem0, isem1, gsem0, gsem1, osem0, osem1):
        wid = lax.axis_index("s") * _NUM_CORES + lax.axis_index("c")
        base = wid * b_per_w
        idx_v = (idx_v0, idx_v1)
        rows_v = (rows_v0, rows_v1)
        isem = (isem0, isem1)
        gsem = (gsem0, gsem1)
        osem = (osem0, osem1)

        def start_idx(j, b):
            pltpu.async_copy(
                idx_hbm.at[pl.ds(base + j * chunk, chunk)], idx_v[b], isem[b])

        start_idx(0, 0)
        if n_chunks > 1:
            start_idx(1, 1)

        for j in range(n_chunks):
            b = j % 2
            pltpu.make_async_copy(
                idx_hbm.at[pl.ds(base + j * chunk, chunk)], idx_v[b],
                isem[b]).wait()
            if j >= 2:
                pltpu.make_async_copy(
                    rows_v[b],
                    out_hbm.at[pl.ds(base + (j - 2) * chunk, chunk)],
                    osem[b]).wait()
            pltpu.async_copy(table_hbm.at[idx_v[b]], rows_v[b], gsem[b])
            pltpu.make_async_copy(
                table_hbm.at[idx_v[b]], rows_v[b], gsem[b]).wait()
            pltpu.async_copy(
                rows_v[b], out_hbm.at[pl.ds(base + j * chunk, chunk)],
                osem[b])
            if j + 2 < n_chunks:
                start_idx(j + 2, b)

        for j in (n_chunks - 2, n_chunks - 1):
            if j >= 0:
                b = j % 2
                pltpu.make_async_copy(
                    rows_v[b], out_hbm.at[pl.ds(base + j * chunk, chunk)],
                    osem[b]).wait()

    return k(idx, table)


def kernel(inputs, embedding_matrix):
    batch, hist = inputs.shape
    idx = inputs.reshape(-1)
    b_per_w = idx.shape[0] // _NUM_WORKERS
    chunk = 1600
    out = _sc_gather(idx, embedding_matrix, chunk=chunk,
                     n_chunks=b_per_w // chunk)
    return out.reshape(batch, hist, _EMBED_DIM)
